# baseline (speedup 1.0000x reference)
import functools

import jax
import jax.numpy as jnp
from jax import lax
from jax.experimental import pallas as pl
from jax.experimental.pallas import tpu as pltpu
from jax.experimental.pallas import tpu_sc as plsc

C = 8
V = 2048
D = 64
LANES = 16

_info = plsc.get_sparse_core_info()
_NC, _NS = _info.num_cores, _info.num_subcores
NW = _NC * _NS


NBUF = 2


@functools.lru_cache(maxsize=None)
def _build(rows):
    rows_per_w = rows // NW
    chunk = 16
    nchunk = rows_per_w // chunk
    nouter = nchunk // NBUF
    mesh = plsc.VectorSubcoreMesh(core_axis_name="c", subcore_axis_name="s")

    @functools.partial(
        pl.kernel,
        mesh=mesh,
        out_type=jax.ShapeDtypeStruct((rows, D), jnp.float32),
        compiler_params=pltpu.CompilerParams(use_tc_tiling_on_sc=False),
        scratch_types=[
            pltpu.VMEM((nchunk, chunk * C), jnp.int32),
            pltpu.VMEM((NBUF, chunk * C, D), jnp.float32),
            pltpu.VMEM((NBUF, chunk, D), jnp.float32),
            pltpu.VMEM((C, LANES), jnp.float32),
            pltpu.VMEM_SHARED((C * V, D), jnp.float32),
            [pltpu.SemaphoreType.DMA] * NBUF,
            [pltpu.SemaphoreType.DMA] * NBUF,
            [pltpu.SemaphoreType.DMA] * NBUF,
        ],
    )
    def k(tok_hbm, table_hbm, scale_hbm, out_hbm,
          idx_v, buf_v, outb_v, scale_v, spt, gsems, gsems2, ssems):
        wid = lax.axis_index("s") * _NC + lax.axis_index("c")
        base = wid * rows_per_w
        sid = lax.axis_index("s")
        tchunk = C * V // _NS
        tstep = chunk * C
        pltpu.sync_copy(scale_hbm, scale_v)
        sv = scale_v[sid // (_NS // C), :]


        cb = sid // (_NS // C)
        def fill_body(p, carry):
            r = sid * tchunk + p * tstep
            loc = r - cb * V
            pltpu.sync_copy(table_hbm.at[cb, pl.ds(loc, tstep)], buf_v.at[0])

            def scale_body(i, carry2):
                for u in range(4):
                    for g in range(D // LANES):
                        sl = pl.ds(g * LANES, LANES)
                        buf_v[0, i * 4 + u, sl] = buf_v[0, i * 4 + u, sl] * sv
                return carry2

            lax.fori_loop(0, tstep // 4, scale_body, 0)
            pltpu.sync_copy(buf_v.at[0], spt.at[pl.ds(r, tstep)])
            return carry

        lax.fori_loop(0, tchunk // tstep, fill_body, 0)
        pltpu.sync_copy(tok_hbm.at[pl.ds(wid * nchunk, nchunk)], idx_v)

        offs = (lax.iota(jnp.int32, LANES) & 7) * V

        def idx_body(j, carry):
            for t in range(chunk * C // LANES):
                sl = pl.ds(t * LANES, LANES)
                idx_v[j, sl] = idx_v[j, sl] + offs
            return carry

        lax.fori_loop(0, nchunk, idx_body, 0)
        plsc.subcore_barrier()

        half = chunk * C // 2

        def gather(j, b):
            pltpu.async_copy(spt.at[idx_v.at[j, pl.ds(0, half)]],
                             buf_v.at[b, pl.ds(0, half)], gsems[b])
            pltpu.async_copy(spt.at[idx_v.at[j, pl.ds(half, half)]],
                             buf_v.at[b, pl.ds(half, half)], gsems2[b])

        for b in range(NBUF):
            gather(b, b)

        def body(m, carry):
            for b in range(NBUF):
                j = m * NBUF + b
                pltpu.make_async_copy(spt.at[idx_v.at[j, pl.ds(0, half)]],
                                      buf_v.at[b, pl.ds(0, half)],
                                      gsems[b]).wait()
                pltpu.make_async_copy(spt.at[idx_v.at[j, pl.ds(half, half)]],
                                      buf_v.at[b, pl.ds(half, half)],
                                      gsems2[b]).wait()

                @pl.when(m > 0)
                def _():
                    pltpu.make_async_copy(
                        outb_v.at[b],
                        out_hbm.at[pl.ds(base + (j - NBUF) * chunk, chunk)],
                        ssems[b]).wait()

                for i in range(chunk):
                    for g in range(D // LANES):
                        acc = buf_v[b, i * C, pl.ds(g * LANES, LANES)]
                        for c in range(1, C):
                            acc = acc + buf_v[b, i * C + c,
                                              pl.ds(g * LANES, LANES)]
                        outb_v[b, i, pl.ds(g * LANES, LANES)] = acc
                pltpu.async_copy(outb_v.at[b],
                                 out_hbm.at[pl.ds(base + j * chunk, chunk)],
                                 ssems[b])

                @pl.when(m < nouter - 1)
                def _():
                    gather(j + NBUF, b)
            return carry

        lax.fori_loop(0, nouter, body, 0)

        for b in range(NBUF):
            j = (nouter - 1) * NBUF + b
            pltpu.make_async_copy(outb_v.at[b],
                                  out_hbm.at[pl.ds(base + j * chunk, chunk)],
                                  ssems[b]).wait()

    return k


def kernel(tokens, tables, level_scale):
    b, l, _ = tokens.shape
    rows = b * l
    tok = tokens.astype(jnp.int32).reshape(rows * C // 128, 128)
    table = tables
    scale_b = jnp.broadcast_to(level_scale.reshape(C, 1), (C, LANES))
    out = _build(rows)(tok, table, scale_b)
    return out.reshape(b, l, D)


# speedup vs baseline: 1.1226x; 1.1226x over previous
import functools

import jax
import jax.numpy as jnp
from jax import lax
from jax.experimental import pallas as pl
from jax.experimental.pallas import tpu as pltpu
from jax.experimental.pallas import tpu_sc as plsc

C = 8
V = 2048
D = 64
LANES = 16

_info = plsc.get_sparse_core_info()
_NC, _NS = _info.num_cores, _info.num_subcores
NW = _NC * _NS


NBUF = 2


@functools.lru_cache(maxsize=None)
def _build(b, l):
    rows = b * l
    rows_per_w = rows // NW
    chunk = l
    nchunk = rows_per_w // chunk
    nouter = nchunk // NBUF
    bpw = b // NW
    mesh = plsc.VectorSubcoreMesh(core_axis_name="c", subcore_axis_name="s")

    @functools.partial(
        pl.kernel,
        mesh=mesh,
        out_type=jax.ShapeDtypeStruct((b, l, D), jnp.float32),
        compiler_params=pltpu.CompilerParams(use_tc_tiling_on_sc=False),
        scratch_types=[
            pltpu.VMEM((nchunk, chunk * C), jnp.int32),
            pltpu.VMEM((NBUF, chunk * C, D), jnp.float32),
            pltpu.VMEM((NBUF, chunk, D), jnp.float32),
            pltpu.VMEM((C, LANES), jnp.float32),
            pltpu.VMEM_SHARED((C * V, D), jnp.float32),
            [pltpu.SemaphoreType.DMA] * NBUF,
            [pltpu.SemaphoreType.DMA] * NBUF,
            [pltpu.SemaphoreType.DMA] * NBUF,
        ],
    )
    def k(tok_hbm, table_hbm, scale_hbm, out_hbm,
          idx_v, buf_v, outb_v, scale_v, spt, gsems, gsems2, ssems):
        wid = lax.axis_index("s") * _NC + lax.axis_index("c")
        sid = lax.axis_index("s")
        tchunk = C * V // _NS
        tstep = chunk * C
        pltpu.sync_copy(scale_hbm, scale_v)
        sv = scale_v[sid // (_NS // C), :]

        cb = sid // (_NS // C)
        nfill = tchunk // tstep
        rem = tchunk - nfill * tstep

        def stage(loc, n):
            pltpu.sync_copy(table_hbm.at[cb, pl.ds(loc, n)],
                            buf_v.at[0, pl.ds(0, n)])

            def scale_body(i, carry2):
                for u in range(4):
                    for g in range(D // LANES):
                        sl = pl.ds(g * LANES, LANES)
                        buf_v[0, i * 4 + u, sl] = buf_v[0, i * 4 + u, sl] * sv
                return carry2

            lax.fori_loop(0, n // 4, scale_body, 0)
            pltpu.sync_copy(buf_v.at[0, pl.ds(0, n)],
                            spt.at[pl.ds(cb * V + loc, n)])

        def fill_body(p, carry):
            stage((sid % (_NS // C)) * tchunk + p * tstep, tstep)
            return carry

        lax.fori_loop(0, nfill, fill_body, 0)
        if rem:
            stage((sid % (_NS // C)) * tchunk + nfill * tstep, rem)
        pltpu.sync_copy(tok_hbm.at[pl.ds(wid * bpw, bpw)], idx_v)

        offs = (lax.iota(jnp.int32, LANES) & 7) * V

        def idx_body(j, carry):
            for t in range(chunk * C // LANES):
                sl = pl.ds(t * LANES, LANES)
                idx_v[j, sl] = idx_v[j, sl] + offs
            return carry

        lax.fori_loop(0, nchunk, idx_body, 0)
        plsc.subcore_barrier()

        half = chunk * C // 2

        def gather(j, bb):
            pltpu.async_copy(spt.at[idx_v.at[j, pl.ds(0, half)]],
                             buf_v.at[bb, pl.ds(0, half)], gsems[bb])
            pltpu.async_copy(spt.at[idx_v.at[j, pl.ds(half, half)]],
                             buf_v.at[bb, pl.ds(half, half)], gsems2[bb])

        for bb in range(NBUF):
            gather(bb, bb)

        def body(m, carry):
            for bb in range(NBUF):
                j = m * NBUF + bb
                pltpu.make_async_copy(spt.at[idx_v.at[j, pl.ds(0, half)]],
                                      buf_v.at[bb, pl.ds(0, half)],
                                      gsems[bb]).wait()
                pltpu.make_async_copy(spt.at[idx_v.at[j, pl.ds(half, half)]],
                                      buf_v.at[bb, pl.ds(half, half)],
                                      gsems2[bb]).wait()

                @pl.when(m > 0)
                def _():
                    pltpu.make_async_copy(
                        outb_v.at[bb],
                        out_hbm.at[wid * bpw + (j - NBUF)],
                        ssems[bb]).wait()

                for i in range(chunk):
                    for g in range(D // LANES):
                        acc = buf_v[bb, i * C, pl.ds(g * LANES, LANES)]
                        for c in range(1, C):
                            acc = acc + buf_v[bb, i * C + c,
                                              pl.ds(g * LANES, LANES)]
                        outb_v[bb, i, pl.ds(g * LANES, LANES)] = acc
                pltpu.async_copy(outb_v.at[bb],
                                 out_hbm.at[wid * bpw + j],
                                 ssems[bb])

                @pl.when(m < nouter - 1)
                def _():
                    gather(j + NBUF, bb)
            return carry

        lax.fori_loop(0, nouter, body, 0)

        for bb in range(NBUF):
            j = (nouter - 1) * NBUF + bb
            pltpu.make_async_copy(outb_v.at[bb],
                                  out_hbm.at[wid * bpw + j],
                                  ssems[bb]).wait()

    return k


def kernel(tokens, tables, level_scale):
    b, l, _ = tokens.shape
    tok = tokens.astype(jnp.int32).reshape(b, l * C)
    scale_b = jnp.broadcast_to(level_scale.reshape(C, 1), (C, LANES))
    return _build(b, l)(tok, tables, scale_b)
